# baseline (speedup 1.0000x reference)
import functools

import jax
import jax.numpy as jnp
from jax.experimental import pallas as pl
from jax.experimental.pallas import tpu as pltpu

N = 10000
NPAD = 10240
E = 320000
HID = 64
EDIM = 16
EPS = 1e-5

NBLK = 128
EBLK = 512


def _prep_body(x_ref, wp_ref, bp_ref, wad_ref, was_ref, ad_ref, as_ref):
    h = jnp.maximum(jnp.dot(x_ref[...], wp_ref[...],
                            preferred_element_type=jnp.float32) + bp_ref[...], 0.0)
    ad_ref[...] = jnp.dot(h, wad_ref[...], preferred_element_type=jnp.float32)
    as_ref[...] = jnp.dot(h, was_ref[...], preferred_element_type=jnp.float32)


def _prep(x, Wp, bp, Wad, Was):
    grid = (NPAD // NBLK,)
    return pl.pallas_call(
        _prep_body,
        grid=grid,
        in_specs=[
            pl.BlockSpec((NBLK, 128), lambda i: (i, 0)),
            pl.BlockSpec((128, HID), lambda i: (0, 0)),
            pl.BlockSpec((1, HID), lambda i: (0, 0)),
            pl.BlockSpec((HID, HID), lambda i: (0, 0)),
            pl.BlockSpec((HID, HID), lambda i: (0, 0)),
        ],
        out_specs=[
            pl.BlockSpec((NBLK, HID), lambda i: (i, 0)),
            pl.BlockSpec((NBLK, HID), lambda i: (i, 0)),
        ],
        out_shape=[
            jax.ShapeDtypeStruct((NPAD, HID), jnp.float32),
            jax.ShapeDtypeStruct((NPAD, HID), jnp.float32),
        ],
    )(x, Wp, bp, Wad, Was)


def _edge_mlp_body(g_ref, e_ref, wae_ref, ba_ref, wb_ref, bb_ref, o_ref):
    pre = g_ref[...] + jnp.dot(e_ref[...], wae_ref[...],
                               preferred_element_type=jnp.float32) + ba_ref[...]
    o_ref[...] = jnp.dot(jnp.maximum(pre, 0.0), wb_ref[...],
                         preferred_element_type=jnp.float32) + bb_ref[...]


def _edge_mlp(G, eattr, Wae, ba, Wb, bb):
    ne = G.shape[0]
    grid = (ne // EBLK,)
    return pl.pallas_call(
        _edge_mlp_body,
        grid=grid,
        in_specs=[
            pl.BlockSpec((EBLK, HID), lambda i: (i, 0)),
            pl.BlockSpec((EBLK, EDIM), lambda i: (i, 0)),
            pl.BlockSpec((EDIM, HID), lambda i: (0, 0)),
            pl.BlockSpec((1, HID), lambda i: (0, 0)),
            pl.BlockSpec((HID, HID), lambda i: (0, 0)),
            pl.BlockSpec((1, HID), lambda i: (0, 0)),
        ],
        out_specs=pl.BlockSpec((EBLK, HID), lambda i: (i, 0)),
        out_shape=jax.ShapeDtypeStruct((ne, HID), jnp.float32),
    )(G, eattr, Wae, ba, Wb, bb)


def _bn_prep_body(agg_ref, g_ref, be_ref, wad_ref, was_ref, ad_ref, as_ref):
    a = agg_ref[...]
    a = jnp.where(jnp.isfinite(a), a, 0.0)
    row = jax.lax.broadcasted_iota(jnp.int32, (NPAD, 1), 0)
    am = jnp.where(row < N, a, 0.0)
    mu = jnp.sum(am, axis=0, keepdims=True) / N
    var = jnp.sum(am * am, axis=0, keepdims=True) / N - mu * mu
    h = jnp.maximum(g_ref[...] * (a - mu) * jax.lax.rsqrt(var + EPS) + be_ref[...], 0.0)
    ad_ref[...] = jnp.dot(h, wad_ref[...], preferred_element_type=jnp.float32)
    as_ref[...] = jnp.dot(h, was_ref[...], preferred_element_type=jnp.float32)


def _bn_prep(agg, g, be, Wad, Was):
    return pl.pallas_call(
        _bn_prep_body,
        out_shape=[
            jax.ShapeDtypeStruct((NPAD, HID), jnp.float32),
            jax.ShapeDtypeStruct((NPAD, HID), jnp.float32),
        ],
    )(agg, g, be, Wad, Was)


def _final_body(agg_ref, g_ref, be_ref, wm1_ref, bm1_ref, wm2_ref, bm2_ref, o_ref):
    a = agg_ref[...]
    a = jnp.where(jnp.isfinite(a), a, 0.0)
    row = jax.lax.broadcasted_iota(jnp.int32, (NPAD, 1), 0)
    am = jnp.where(row < N, a, 0.0)
    mu = jnp.sum(am, axis=0, keepdims=True) / N
    var = jnp.sum(am * am, axis=0, keepdims=True) / N - mu * mu
    h = jnp.maximum(g_ref[...] * (a - mu) * jax.lax.rsqrt(var + EPS) + be_ref[...], 0.0)
    t = jnp.maximum(jnp.dot(h, wm1_ref[...], preferred_element_type=jnp.float32) + bm1_ref[...], 0.0)
    o_ref[...] = jnp.dot(t, wm2_ref[...], preferred_element_type=jnp.float32) + bm2_ref[...]


def _final(agg, g, be, Wm1, bm1, Wm2, bm2):
    return pl.pallas_call(
        _final_body,
        out_shape=jax.ShapeDtypeStruct((NPAD, HID), jnp.float32),
    )(agg, g, be, Wm1, bm1, Wm2, bm2)


def kernel(x, edge_index, edge_attr, Wp, bp, W0a, b0a, W0b, b0b, g0, be0,
           W1a, b1a, W1b, b1b, g1, be1, Wm1, bm1, Wm2, bm2):
    src = edge_index[0]
    dst = edge_index[1]

    xpad = jnp.pad(x, ((0, NPAD - N), (0, 0)))
    bp2 = bp.reshape(1, HID)

    W0ad, W0as, W0ae = W0a[:HID], W0a[HID:2 * HID], W0a[2 * HID:]
    W1ad, W1as, W1ae = W1a[:HID], W1a[HID:2 * HID], W1a[2 * HID:]

    Ad0, As0 = _prep(xpad, Wp, bp2, W0ad, W0as)
    G0 = Ad0[dst] + As0[src]
    M0 = _edge_mlp(G0, edge_attr, W0ae, b0a.reshape(1, HID), W0b, b0b.reshape(1, HID))
    agg0 = jax.ops.segment_max(M0, dst, num_segments=NPAD)

    Ad1, As1 = _bn_prep(agg0, g0.reshape(1, HID), be0.reshape(1, HID), W1ad, W1as)
    G1 = Ad1[dst] + As1[src]
    M1 = _edge_mlp(G1, edge_attr, W1ae, b1a.reshape(1, HID), W1b, b1b.reshape(1, HID))
    agg1 = jax.ops.segment_max(M1, dst, num_segments=NPAD)

    out = _final(agg1, g1.reshape(1, HID), be1.reshape(1, HID),
                 Wm1, bm1.reshape(1, HID), Wm2, bm2.reshape(1, HID))
    return out[:N]


# speedup vs baseline: 1.4646x; 1.4646x over previous
import functools

import jax
import jax.numpy as jnp
from jax import lax
from jax.experimental import pallas as pl
from jax.experimental.pallas import tpu as pltpu
from jax.experimental.pallas import tpu_sc as plsc

N = 10000
NPAD = 10240
E = 320000
HID = 64
EDIM = 16
EPS = 1e-5

NBLK = 128
EBLK = 512

NWORK = 32
CH = 128
CPW = 80
EPAD = NWORK * CPW * CH


def _sc_gather_add(Ad, As, dst2d, src2d):
    mesh = plsc.VectorSubcoreMesh(core_axis_name="c", subcore_axis_name="s")

    @functools.partial(
        pl.kernel,
        out_type=jax.ShapeDtypeStruct((EPAD, HID), jnp.float32),
        mesh=mesh,
        scratch_types=[
            pltpu.VMEM((CPW, CH), jnp.int32),
            pltpu.VMEM((CPW, CH), jnp.int32),
            pltpu.VMEM((CH, HID), jnp.float32),
            pltpu.VMEM((CH, HID), jnp.float32),
            pltpu.SemaphoreType.DMA,
            pltpu.SemaphoreType.DMA,
        ],
        compiler_params=pltpu.CompilerParams(use_tc_tiling_on_sc=False),
    )
    def k(ad_hbm, as_hbm, d_hbm, s_hbm, out_hbm, didx, sidx, bufa, bufb,
          sema, semb):
        wid = lax.axis_index("s") * 2 + lax.axis_index("c")
        row0 = wid * CPW
        pltpu.sync_copy(d_hbm.at[pl.ds(row0, CPW)], didx)
        pltpu.sync_copy(s_hbm.at[pl.ds(row0, CPW)], sidx)

        def chunk_body(j, carry):
            ca = pltpu.async_copy(ad_hbm.at[didx.at[j]], bufa, sema)
            cb = pltpu.async_copy(as_hbm.at[sidx.at[j]], bufb, semb)
            ca.wait()
            cb.wait()

            def add_body(i, c2):
                for c in range(HID // 16):
                    sl = pl.ds(c * 16, 16)
                    bufa[i, sl] = bufa[i, sl] + bufb[i, sl]
                return c2

            lax.fori_loop(0, CH, add_body, 0, unroll=2)
            pltpu.sync_copy(bufa, out_hbm.at[pl.ds((row0 + j) * CH, CH)])
            return carry

        lax.fori_loop(0, CPW, chunk_body, 0)

    return k(Ad, As, dst2d, src2d)


def _prep_body(x_ref, wp_ref, bp_ref, wad_ref, was_ref, ad_ref, as_ref):
    h = jnp.maximum(jnp.dot(x_ref[...], wp_ref[...],
                            preferred_element_type=jnp.float32) + bp_ref[...], 0.0)
    ad_ref[...] = jnp.dot(h, wad_ref[...], preferred_element_type=jnp.float32)
    as_ref[...] = jnp.dot(h, was_ref[...], preferred_element_type=jnp.float32)


def _prep(x, Wp, bp, Wad, Was):
    grid = (NPAD // NBLK,)
    return pl.pallas_call(
        _prep_body,
        grid=grid,
        in_specs=[
            pl.BlockSpec((NBLK, 128), lambda i: (i, 0)),
            pl.BlockSpec((128, HID), lambda i: (0, 0)),
            pl.BlockSpec((1, HID), lambda i: (0, 0)),
            pl.BlockSpec((HID, HID), lambda i: (0, 0)),
            pl.BlockSpec((HID, HID), lambda i: (0, 0)),
        ],
        out_specs=[
            pl.BlockSpec((NBLK, HID), lambda i: (i, 0)),
            pl.BlockSpec((NBLK, HID), lambda i: (i, 0)),
        ],
        out_shape=[
            jax.ShapeDtypeStruct((NPAD, HID), jnp.float32),
            jax.ShapeDtypeStruct((NPAD, HID), jnp.float32),
        ],
    )(x, Wp, bp, Wad, Was)


def _edge_mlp_body(g_ref, e_ref, wae_ref, ba_ref, wb_ref, bb_ref, o_ref):
    pre = g_ref[...] + jnp.dot(e_ref[...], wae_ref[...],
                               preferred_element_type=jnp.float32) + ba_ref[...]
    o_ref[...] = jnp.dot(jnp.maximum(pre, 0.0), wb_ref[...],
                         preferred_element_type=jnp.float32) + bb_ref[...]


def _edge_mlp(G, eattr, Wae, ba, Wb, bb):
    ne = G.shape[0]
    grid = (ne // EBLK,)
    return pl.pallas_call(
        _edge_mlp_body,
        grid=grid,
        in_specs=[
            pl.BlockSpec((EBLK, HID), lambda i: (i, 0)),
            pl.BlockSpec((EBLK, EDIM), lambda i: (i, 0)),
            pl.BlockSpec((EDIM, HID), lambda i: (0, 0)),
            pl.BlockSpec((1, HID), lambda i: (0, 0)),
            pl.BlockSpec((HID, HID), lambda i: (0, 0)),
            pl.BlockSpec((1, HID), lambda i: (0, 0)),
        ],
        out_specs=pl.BlockSpec((EBLK, HID), lambda i: (i, 0)),
        out_shape=jax.ShapeDtypeStruct((ne, HID), jnp.float32),
    )(G, eattr, Wae, ba, Wb, bb)


def _bn_prep_body(agg_ref, g_ref, be_ref, wad_ref, was_ref, ad_ref, as_ref):
    a = agg_ref[...]
    a = jnp.where(jnp.isfinite(a), a, 0.0)
    row = jax.lax.broadcasted_iota(jnp.int32, (NPAD, 1), 0)
    am = jnp.where(row < N, a, 0.0)
    mu = jnp.sum(am, axis=0, keepdims=True) / N
    var = jnp.sum(am * am, axis=0, keepdims=True) / N - mu * mu
    h = jnp.maximum(g_ref[...] * (a - mu) * jax.lax.rsqrt(var + EPS) + be_ref[...], 0.0)
    ad_ref[...] = jnp.dot(h, wad_ref[...], preferred_element_type=jnp.float32)
    as_ref[...] = jnp.dot(h, was_ref[...], preferred_element_type=jnp.float32)


def _bn_prep(agg, g, be, Wad, Was):
    return pl.pallas_call(
        _bn_prep_body,
        out_shape=[
            jax.ShapeDtypeStruct((NPAD, HID), jnp.float32),
            jax.ShapeDtypeStruct((NPAD, HID), jnp.float32),
        ],
    )(agg, g, be, Wad, Was)


def _final_body(agg_ref, g_ref, be_ref, wm1_ref, bm1_ref, wm2_ref, bm2_ref, o_ref):
    a = agg_ref[...]
    a = jnp.where(jnp.isfinite(a), a, 0.0)
    row = jax.lax.broadcasted_iota(jnp.int32, (NPAD, 1), 0)
    am = jnp.where(row < N, a, 0.0)
    mu = jnp.sum(am, axis=0, keepdims=True) / N
    var = jnp.sum(am * am, axis=0, keepdims=True) / N - mu * mu
    h = jnp.maximum(g_ref[...] * (a - mu) * jax.lax.rsqrt(var + EPS) + be_ref[...], 0.0)
    t = jnp.maximum(jnp.dot(h, wm1_ref[...], preferred_element_type=jnp.float32) + bm1_ref[...], 0.0)
    o_ref[...] = jnp.dot(t, wm2_ref[...], preferred_element_type=jnp.float32) + bm2_ref[...]


def _final(agg, g, be, Wm1, bm1, Wm2, bm2):
    return pl.pallas_call(
        _final_body,
        out_shape=jax.ShapeDtypeStruct((NPAD, HID), jnp.float32),
    )(agg, g, be, Wm1, bm1, Wm2, bm2)


def kernel(x, edge_index, edge_attr, Wp, bp, W0a, b0a, W0b, b0b, g0, be0,
           W1a, b1a, W1b, b1b, g1, be1, Wm1, bm1, Wm2, bm2):
    src = edge_index[0]
    dst = edge_index[1]

    xpad = jnp.pad(x, ((0, NPAD - N), (0, 0)))
    bp2 = bp.reshape(1, HID)

    dst_pad = jnp.pad(dst, (0, EPAD - E), constant_values=NPAD - 1)
    src_pad = jnp.pad(src, (0, EPAD - E), constant_values=NPAD - 1)
    dst2d = dst_pad.reshape(EPAD // CH, CH)
    src2d = src_pad.reshape(EPAD // CH, CH)
    eattr_pad = jnp.pad(edge_attr, ((0, EPAD - E), (0, 0)))

    W0ad, W0as, W0ae = W0a[:HID], W0a[HID:2 * HID], W0a[2 * HID:]
    W1ad, W1as, W1ae = W1a[:HID], W1a[HID:2 * HID], W1a[2 * HID:]

    Ad0, As0 = _prep(xpad, Wp, bp2, W0ad, W0as)
    G0 = _sc_gather_add(Ad0, As0, dst2d, src2d)
    M0 = _edge_mlp(G0, eattr_pad, W0ae, b0a.reshape(1, HID), W0b, b0b.reshape(1, HID))
    agg0 = jax.ops.segment_max(M0, dst_pad, num_segments=NPAD)

    Ad1, As1 = _bn_prep(agg0, g0.reshape(1, HID), be0.reshape(1, HID), W1ad, W1as)
    G1 = _sc_gather_add(Ad1, As1, dst2d, src2d)
    M1 = _edge_mlp(G1, eattr_pad, W1ae, b1a.reshape(1, HID), W1b, b1b.reshape(1, HID))
    agg1 = jax.ops.segment_max(M1, dst_pad, num_segments=NPAD)

    out = _final(agg1, g1.reshape(1, HID), be1.reshape(1, HID),
                 Wm1, bm1.reshape(1, HID), Wm2, bm2.reshape(1, HID))
    return out[:N]
